# baseline (speedup 1.0000x reference)
import functools
import math

import jax
import jax.numpy as jnp
from jax import lax
from jax.experimental import pallas as pl
from jax.experimental.pallas import tpu as pltpu
from jax.experimental.pallas import tpu_sc as plsc

B = 1024
L = 200
D = 512
NV = 65
NC = 11
REP = 34

_NCORES = 2
_NSUB = 16
_NW = _NCORES * _NSUB
_ROWS_W = B * L // _NW
_CH = 32
_NCHUNK = _ROWS_W // _CH


def _tab_body(ct_ref, aa_ref, pc_ref, rep_ref):
    rows = NC * L
    d_idx = lax.broadcasted_iota(jnp.int32, (rows, D), 1)
    r_idx = lax.broadcasted_iota(jnp.int32, (rows, D), 0)
    pos = (r_idx % L).astype(jnp.float32)
    d_even = ((d_idx // 2) * 2).astype(jnp.float32)
    ang = pos * jnp.exp(d_even * (-math.log(10000.0) / D))
    pe = jnp.where(d_idx % 2 == 0, jnp.sin(ang), jnp.cos(ang))
    c1 = lax.broadcasted_iota(jnp.int32, (rows, NC), 0) // L
    oh = (c1 == lax.broadcasted_iota(jnp.int32, (rows, NC), 1)
          ).astype(jnp.float32)
    ch = lax.dot_general(oh, ct_ref[...], (((1,), (0,)), ((), ())),
                         preferred_element_type=jnp.float32)
    pc_ref[...] = pe + ch
    aa = aa_ref[...]
    for k in range(REP):
        rep_ref[k * NV:(k + 1) * NV, :] = aa


def _build_tabs(aa_table, charge_table):
    return pl.pallas_call(
        _tab_body,
        out_shape=[
            jax.ShapeDtypeStruct((NC * L, D), jnp.float32),
            jax.ShapeDtypeStruct((REP * NV, D), jnp.float32),
        ],
    )(charge_table, aa_table)


def _sc_body(tok_hbm, chg_hbm, rep_hbm, pc_hbm, out_hbm,
             charges_v, tok_v, pcidx_v, aaidx_v, buf_v, buf2_v,
             sem_pc, sem_aa, sem_w):
    cid = lax.axis_index("c")
    sid = lax.axis_index("s")
    wid = sid * _NCORES + cid
    base = wid * _ROWS_W

    pltpu.sync_copy(chg_hbm, charges_v)
    pltpu.sync_copy(tok_hbm.at[pl.ds(base, _ROWS_W)], tok_v)

    @plsc.parallel_loop(0, _ROWS_W // 16, unroll=4)
    def _(j):
        loc = j * 16 + lax.iota(jnp.int32, 16)
        row = base + loc
        cvec = plsc.load_gather(charges_v, [row // L])
        pcidx_v[pl.ds(j * 16, 16)] = cvec * L + row % L
        tvec = plsc.load_gather(tok_v, [loc])
        aaidx_v[pl.ds(j * 16, 16)] = ((wid + loc) % REP) * NV + tvec

    def pair(g, carry):
        offs = [(2 * g + s) * _CH for s in range(2)]
        descs_pc = []
        descs_aa = []
        for s in range(2):
            descs_pc.append(pltpu.async_copy(
                pc_hbm.at[pcidx_v.at[pl.ds(offs[s], _CH)]],
                buf_v.at[s], sem_pc.at[s]))
            descs_aa.append(pltpu.async_copy(
                rep_hbm.at[aaidx_v.at[pl.ds(offs[s], _CH)]],
                buf2_v.at[s], sem_aa.at[s]))
        descs_w = []
        for s in range(2):
            descs_pc[s].wait()
            descs_aa[s].wait()

            @plsc.parallel_loop(0, _CH, unroll=2)
            def _(r, s=s):
                for k in range(D // 16):
                    sl = pl.ds(k * 16, 16)
                    plsc.addupdate(buf_v.at[s, r, sl], buf2_v[s, r, sl])

            descs_w.append(pltpu.async_copy(
                buf_v.at[s], out_hbm.at[pl.ds(base + offs[s], _CH)],
                sem_w.at[s]))
        for s in range(2):
            descs_w[s].wait()
        return carry

    lax.fori_loop(0, _NCHUNK // 2, pair, 0)


def kernel(tokens, charges, aa_table, charge_table):
    pc, aa_rep = _build_tabs(aa_table, charge_table)
    mesh = plsc.VectorSubcoreMesh(
        core_axis_name="c", subcore_axis_name="s",
        num_cores=_NCORES, num_subcores=_NSUB)
    sc = functools.partial(
        pl.kernel,
        out_type=jax.ShapeDtypeStruct((B * L, D), jnp.float32),
        mesh=mesh,
        compiler_params=pltpu.CompilerParams(needs_layout_passes=False),
        scratch_types=[
            pltpu.VMEM((B,), jnp.int32),
            pltpu.VMEM((_ROWS_W,), jnp.int32),
            pltpu.VMEM((_ROWS_W,), jnp.int32),
            pltpu.VMEM((_ROWS_W,), jnp.int32),
            pltpu.VMEM((2, _CH, D), jnp.float32),
            pltpu.VMEM((2, _CH, D), jnp.float32),
            pltpu.SemaphoreType.DMA((2,)),
            pltpu.SemaphoreType.DMA((2,)),
            pltpu.SemaphoreType.DMA((2,)),
        ],
    )(_sc_body)
    out2 = sc(tokens.reshape(B * L), charges, aa_rep, pc)
    return out2.reshape(B, L, D)


# speedup vs baseline: 1.2126x; 1.2126x over previous
import functools
import math

import jax
import jax.numpy as jnp
from jax import lax
from jax.experimental import pallas as pl
from jax.experimental.pallas import tpu as pltpu
from jax.experimental.pallas import tpu_sc as plsc

B = 1024
L = 200
D = 512
NV = 65
NC = 11
NT = NV + NC
REP = 34

S = 153600
BLK = 3200

_NCORES = 2
_NSUB = 16
_NW = _NCORES * _NSUB
_SC_ROWS = B * L - S
_ROWS_W = _SC_ROWS // _NW
_CH = 32
_NCHUNK = _ROWS_W // _CH


def _tc_body(fidx_ref, tab_ref, out_ref, pe_ref):
    @pl.when(pl.program_id(0) == 0)
    def _():
        d_idx = lax.broadcasted_iota(jnp.int32, (L, D), 1)
        pos = lax.broadcasted_iota(jnp.int32, (L, D), 0).astype(jnp.float32)
        d_even = ((d_idx // 2) * 2).astype(jnp.float32)
        ang = pos * jnp.exp(d_even * (-math.log(10000.0) / D))
        pe = jnp.where(d_idx % 2 == 0, jnp.sin(ang), jnp.cos(ang))
        for k in range(BLK // L):
            pe_ref[k * L:(k + 1) * L, :] = pe

    tok = fidx_ref[:, 0:1]
    cid = fidx_ref[:, 1:2]
    cols = lax.broadcasted_iota(jnp.int32, (BLK, NT), 1)
    oh = (((tok == cols) & (tok != 0)) | (cid == cols)).astype(jnp.float32)
    aa_ch = lax.dot_general(oh, tab_ref[...], (((1,), (0,)), ((), ())),
                            preferred_element_type=jnp.float32)
    out_ref[...] = aa_ch + pe_ref[...]


def _tc_part(fidx, tab):
    return pl.pallas_call(
        _tc_body,
        grid=(S // BLK,),
        in_specs=[
            pl.BlockSpec((BLK, 2), lambda i: (i, 0)),
            pl.BlockSpec((NT, D), lambda i: (0, 0)),
        ],
        out_specs=pl.BlockSpec((BLK, D), lambda i: (i, 0)),
        out_shape=jax.ShapeDtypeStruct((S, D), jnp.float32),
        scratch_shapes=[pltpu.VMEM((BLK, D), jnp.float32)],
    )(fidx, tab)


def _tab_body(ct_ref, aa_ref, pc_ref, rep_ref):
    rows = NC * L
    d_idx = lax.broadcasted_iota(jnp.int32, (rows, D), 1)
    r_idx = lax.broadcasted_iota(jnp.int32, (rows, D), 0)
    pos = (r_idx % L).astype(jnp.float32)
    d_even = ((d_idx // 2) * 2).astype(jnp.float32)
    ang = pos * jnp.exp(d_even * (-math.log(10000.0) / D))
    pe = jnp.where(d_idx % 2 == 0, jnp.sin(ang), jnp.cos(ang))
    c1 = lax.broadcasted_iota(jnp.int32, (rows, NC), 0) // L
    oh = (c1 == lax.broadcasted_iota(jnp.int32, (rows, NC), 1)
          ).astype(jnp.float32)
    ch = lax.dot_general(oh, ct_ref[...], (((1,), (0,)), ((), ())),
                         preferred_element_type=jnp.float32)
    pc_ref[...] = pe + ch
    aa = aa_ref[...]
    for k in range(REP):
        rep_ref[k * NV:(k + 1) * NV, :] = aa


def _build_tabs(aa_table, charge_table):
    return pl.pallas_call(
        _tab_body,
        out_shape=[
            jax.ShapeDtypeStruct((NC * L, D), jnp.float32),
            jax.ShapeDtypeStruct((REP * NV, D), jnp.float32),
        ],
    )(charge_table, aa_table)


def _sc_body(tok_hbm, chg_hbm, rep_hbm, pc_hbm, out_hbm,
             charges_v, tok_v, pcidx_v, aaidx_v, buf_v, buf2_v,
             sem_pc, sem_aa, sem_w):
    cid = lax.axis_index("c")
    sid = lax.axis_index("s")
    wid = sid * _NCORES + cid
    gbase = S + wid * _ROWS_W

    pltpu.sync_copy(chg_hbm, charges_v)
    pltpu.sync_copy(tok_hbm.at[pl.ds(gbase, _ROWS_W)], tok_v)

    @plsc.parallel_loop(0, _ROWS_W // 16, unroll=4)
    def _(j):
        loc = j * 16 + lax.iota(jnp.int32, 16)
        row = gbase + loc
        cvec = plsc.load_gather(charges_v, [row // L])
        pcidx_v[pl.ds(j * 16, 16)] = cvec * L + row % L
        tvec = plsc.load_gather(tok_v, [loc])
        aaidx_v[pl.ds(j * 16, 16)] = ((wid + loc) % REP) * NV + tvec

    def pair(g, carry):
        offs = [(2 * g + s) * _CH for s in range(2)]
        descs_pc = []
        descs_aa = []
        for s in range(2):
            descs_pc.append(pltpu.async_copy(
                pc_hbm.at[pcidx_v.at[pl.ds(offs[s], _CH)]],
                buf_v.at[s], sem_pc.at[s]))
            descs_aa.append(pltpu.async_copy(
                rep_hbm.at[aaidx_v.at[pl.ds(offs[s], _CH)]],
                buf2_v.at[s], sem_aa.at[s]))
        descs_w = []
        for s in range(2):
            descs_pc[s].wait()
            descs_aa[s].wait()

            @plsc.parallel_loop(0, _CH, unroll=2)
            def _(r, s=s):
                for k in range(D // 16):
                    sl = pl.ds(k * 16, 16)
                    plsc.addupdate(buf_v.at[s, r, sl], buf2_v[s, r, sl])

            descs_w.append(pltpu.async_copy(
                buf_v.at[s], out_hbm.at[pl.ds(wid * _ROWS_W + offs[s], _CH)],
                sem_w.at[s]))
        for s in range(2):
            descs_w[s].wait()
        return carry

    lax.fori_loop(0, _NCHUNK // 2, pair, 0)


def _sc_part(tokens_flat, charges, aa_rep, pc):
    mesh = plsc.VectorSubcoreMesh(
        core_axis_name="c", subcore_axis_name="s",
        num_cores=_NCORES, num_subcores=_NSUB)
    sc = functools.partial(
        pl.kernel,
        out_type=jax.ShapeDtypeStruct((_SC_ROWS, D), jnp.float32),
        mesh=mesh,
        compiler_params=pltpu.CompilerParams(needs_layout_passes=False),
        scratch_types=[
            pltpu.VMEM((B,), jnp.int32),
            pltpu.VMEM((_ROWS_W,), jnp.int32),
            pltpu.VMEM((_ROWS_W,), jnp.int32),
            pltpu.VMEM((_ROWS_W,), jnp.int32),
            pltpu.VMEM((2, _CH, D), jnp.float32),
            pltpu.VMEM((2, _CH, D), jnp.float32),
            pltpu.SemaphoreType.DMA((2,)),
            pltpu.SemaphoreType.DMA((2,)),
            pltpu.SemaphoreType.DMA((2,)),
        ],
    )(_sc_body)
    return sc(tokens_flat, charges, aa_rep, pc)


def kernel(tokens, charges, aa_table, charge_table):
    tokens_flat = tokens.reshape(B * L)
    fidx = jnp.stack(
        [tokens_flat[:S],
         NV + jnp.broadcast_to(charges[:, None], (B, L)).reshape(B * L)[:S]],
        axis=1)
    tab = jnp.concatenate([aa_table, charge_table], axis=0)
    pc, aa_rep = _build_tabs(aa_table, charge_table)
    out_tc = _tc_part(fidx, tab)
    out_sc = _sc_part(tokens_flat, charges, aa_rep, pc)
    return jnp.concatenate([out_tc, out_sc], axis=0).reshape(B, L, D)
